# baseline (speedup 1.0000x reference)
import functools

import jax
import jax.numpy as jnp
from jax import lax
from jax.experimental import pallas as pl
from jax.experimental.pallas import tpu as pltpu
from jax.experimental.pallas import tpu_sc as plsc


def _fused_body(we_ref, x_ref, wg_ref, be_ref, su_ref, lt_ref,
                w4_ref, cnt_ref, bs_ref, *, t, kwe, kpe):
    k = pl.program_id(0)

    @pl.when(k == 0)
    def _init():
        w4_ref[0:2] = jnp.transpose(wg_ref[...])
        w4_ref[2:4] = jnp.zeros_like(w4_ref[2:4])
        cnt_ref[0] = 0.0
        bs_ref[0] = jnp.sum(be_ref[0:1, :])
        bs_ref[1] = jnp.sum(be_ref[1:2, :])

    @pl.when(k < kwe)
    def _accum():
        e = k // kpe
        part = jnp.sum(we_ref[0], axis=0, keepdims=True)
        w4_ref[pl.ds(2 + e, 1)] += part

    @pl.when(k >= kwe)
    def _tokens():
        lt = jax.lax.dot_general(
            w4_ref[...], x_ref[...],
            dimension_numbers=(((1,), (1,)), ((), ())),
            preferred_element_type=jnp.float32)
        m = (lt[1:2] > lt[0:1]).astype(jnp.float32)
        excl = jnp.dot(m, su_ref[...], preferred_element_type=jnp.float32)
        lt_ref[0:2] = lt[0:2]
        lt_ref[2:3] = lt[2:3] + bs_ref[0]
        lt_ref[3:4] = lt[3:4] + bs_ref[1]
        lt_ref[4:5] = excl + cnt_ref[0]
        cnt_ref[0] += jnp.sum(m)


def _sc_route_body(lt_hbm, out_hbm, ltv, valv, *, cap, per_w, nc):
    wid = lax.axis_index("s") * nc + lax.axis_index("c")
    base = wid * per_w

    pltpu.sync_copy(lt_hbm.at[:, pl.ds(base, per_w)], ltv)

    i16 = lax.iota(jnp.int32, 16)
    capf = jnp.float32(cap)
    one = jnp.ones((16,), jnp.float32)
    zero = jnp.zeros((16,), jnp.float32)
    for j in range(per_w // 16):
        sl = pl.ds(16 * j, 16)
        l0 = ltv[0, sl]
        l1 = ltv[1, sl]
        t0 = ltv[2, sl]
        t1 = ltv[3, sl]
        pos1 = ltv[4, sl]
        e1 = l1 > l0
        slin = (base + 16 * j + i16).astype(jnp.float32)
        pos0 = slin - pos1
        pos = jnp.where(e1, pos1, pos0)
        within = jnp.where(pos < capf, one, zero)
        gate = 1.0 / (1.0 + jnp.exp(-jnp.abs(l1 - l0)))
        tsel = jnp.where(e1, t1, t0)
        valv[sl] = gate * within * tsel

    pltpu.sync_copy(valv, out_hbm.at[pl.ds(base, per_w)])


def _lsm_body(v_ref, out_ref):
    v = v_ref[...]
    mx = jnp.max(v, axis=1, keepdims=True)
    lse = jnp.log(jnp.sum(jnp.exp(v - mx), axis=1, keepdims=True)) + mx
    out_ref[...] = v - lse


def kernel(input, wg, We, be):
    B, SEQ, D = input.shape
    E = wg.shape[1]
    S = B * SEQ
    cap = -(-S // E)

    x = input.reshape(S, D)

    F = 512
    KPE = D // F
    KWE = E * KPE
    T = 512
    C = S // T

    ii = lax.broadcasted_iota(jnp.int32, (T, T), 0)
    jj = lax.broadcasted_iota(jnp.int32, (T, T), 1)
    su = (ii < jj).astype(jnp.float32)

    lt = pl.pallas_call(
        functools.partial(_fused_body, t=T, kwe=KWE, kpe=KPE),
        grid=(KWE + C,),
        in_specs=[
            pl.BlockSpec((1, F, D),
                         lambda k: (jnp.minimum(k, KWE - 1) // KPE,
                                    jnp.minimum(k, KWE - 1) % KPE, 0)),
            pl.BlockSpec((T, D), lambda k: (jnp.maximum(k - KWE, 0), 0)),
            pl.BlockSpec((D, E), lambda k: (0, 0)),
            pl.BlockSpec((E, D), lambda k: (0, 0)),
            pl.BlockSpec((T, T), lambda k: (0, 0)),
        ],
        out_specs=pl.BlockSpec((5, T), lambda k: (0, jnp.maximum(k - KWE, 0))),
        out_shape=jax.ShapeDtypeStruct((5, S), jnp.float32),
        scratch_shapes=[
            pltpu.VMEM((4, D), jnp.float32),
            pltpu.SMEM((1,), jnp.float32),
            pltpu.SMEM((2,), jnp.float32),
        ],
    )(We, x, wg, be, su)

    NW = 32
    NC = 2
    per_w = S // NW

    sc_route = functools.partial(
        pl.kernel,
        mesh=plsc.VectorSubcoreMesh(core_axis_name="c", subcore_axis_name="s"),
        out_type=jax.ShapeDtypeStruct((S,), jnp.float32),
        scratch_types=[
            pltpu.VMEM((5, per_w), jnp.float32),
            pltpu.VMEM((per_w,), jnp.float32),
        ],
    )(functools.partial(_sc_route_body, cap=cap, per_w=per_w, nc=NC))

    val = sc_route(lt)

    v = val.reshape(B, SEQ)

    out = pl.pallas_call(
        _lsm_body,
        in_specs=[pl.BlockSpec((B, SEQ), lambda: (0, 0))],
        out_specs=pl.BlockSpec((B, SEQ), lambda: (0, 0)),
        out_shape=jax.ShapeDtypeStruct((B, SEQ), jnp.float32),
    )(v)
    return out


# speedup vs baseline: 1.4818x; 1.4818x over previous
import functools

import jax
import jax.numpy as jnp
from jax.experimental import pallas as pl
from jax.experimental.pallas import tpu as pltpu


def _fused_body(we_hbm, x_hbm, wg_ref, be_ref, su_ref, out_ref,
                webuf, xbuf, w4_ref, sem_we, sem_x, *, cap, t, f, kwe, c):
    d = wg_ref.shape[0]

    def we_copy(i):
        return pltpu.make_async_copy(
            we_hbm.at[i // (kwe // 2), pl.ds((i % (kwe // 2)) * f, f), :],
            webuf.at[i % 2], sem_we.at[i % 2])

    def x_copy(j):
        return pltpu.make_async_copy(
            x_hbm.at[pl.ds(j * t, t), :], xbuf.at[j % 2], sem_x.at[j % 2])

    we_copy(0).start()
    we_copy(1).start()
    x_copy(0).start()
    x_copy(1).start()

    w4_ref[0:2] = jnp.transpose(wg_ref[...])
    w4_ref[2:4] = jnp.zeros_like(w4_ref[2:4])
    bs0 = jnp.sum(be_ref[0:1, :])
    bs1 = jnp.sum(be_ref[1:2, :])

    for i in range(kwe):
        we_copy(i).wait()
        part = jnp.sum(webuf[i % 2], axis=0, keepdims=True)
        w4_ref[pl.ds(2 + i // (kwe // 2), 1)] += part
        if i + 2 < kwe:
            we_copy(i + 2).start()

    cnt = jnp.float32(0.0)
    for j in range(c):
        x_copy(j).wait()
        lt = jax.lax.dot_general(
            w4_ref[...], xbuf[j % 2],
            dimension_numbers=(((1,), (1,)), ((), ())),
            preferred_element_type=jnp.float32)
        if j + 2 < c:
            x_copy(j + 2).start()
        l0, l1, t0, t1 = lt[0:1], lt[1:2], lt[2:3], lt[3:4]
        e1 = l1 > l0
        m = e1.astype(jnp.float32)
        excl = jnp.dot(m, su_ref[...], preferred_element_type=jnp.float32)
        pos1 = excl + cnt
        slin = (j * t + jax.lax.broadcasted_iota(jnp.int32, (1, t), 1)
                ).astype(jnp.float32)
        pos0 = slin - pos1
        pos = jnp.where(e1, pos1, pos0)
        within = (pos < cap).astype(jnp.float32)
        gate = jax.nn.sigmoid(jnp.abs(l1 - l0))
        tsel = jnp.where(e1, t1 + bs1, t0 + bs0)
        out_ref[0:1, pl.ds(j * t, t)] = gate * within * tsel
        cnt = cnt + jnp.sum(m)


def _lsm_body(v_ref, out_ref):
    v = v_ref[...]
    mx = jnp.max(v, axis=1, keepdims=True)
    lse = jnp.log(jnp.sum(jnp.exp(v - mx), axis=1, keepdims=True)) + mx
    out_ref[...] = v - lse


def kernel(input, wg, We, be):
    B, SEQ, D = input.shape
    E = wg.shape[1]
    S = B * SEQ
    cap = -(-S // E)

    x = input.reshape(S, D)

    F = 512
    KWE = E * (D // F)
    T = 512
    C = S // T

    ii = jax.lax.broadcasted_iota(jnp.int32, (T, T), 0)
    jj = jax.lax.broadcasted_iota(jnp.int32, (T, T), 1)
    su = (ii < jj).astype(jnp.float32)

    val = pl.pallas_call(
        functools.partial(_fused_body, cap=float(cap), t=T, f=F,
                          kwe=KWE, c=C),
        in_specs=[
            pl.BlockSpec(memory_space=pl.ANY),
            pl.BlockSpec(memory_space=pl.ANY),
            pl.BlockSpec((D, E), lambda: (0, 0)),
            pl.BlockSpec((E, D), lambda: (0, 0)),
            pl.BlockSpec((T, T), lambda: (0, 0)),
        ],
        out_specs=pl.BlockSpec((1, S), lambda: (0, 0)),
        out_shape=jax.ShapeDtypeStruct((1, S), jnp.float32),
        scratch_shapes=[
            pltpu.VMEM((2, F, D), jnp.float32),
            pltpu.VMEM((2, T, D), jnp.float32),
            pltpu.VMEM((4, D), jnp.float32),
            pltpu.SemaphoreType.DMA((2,)),
            pltpu.SemaphoreType.DMA((2,)),
        ],
    )(We, x, wg, be, su)

    v = val.reshape(B, SEQ)

    out = pl.pallas_call(
        _lsm_body,
        in_specs=[pl.BlockSpec((B, SEQ), lambda: (0, 0))],
        out_specs=pl.BlockSpec((B, SEQ), lambda: (0, 0)),
        out_shape=jax.ShapeDtypeStruct((B, SEQ), jnp.float32),
    )(v)
    return out


# speedup vs baseline: 1.5471x; 1.0441x over previous
import functools

import jax
import jax.numpy as jnp
from jax.experimental import pallas as pl
from jax.experimental.pallas import tpu as pltpu


def _fused_body(we_hbm, x_hbm, wg_ref, be_ref, su_ref, out_ref,
                webuf, xbuf, w4_ref, sem_we, sem_x, *, cap, t, f, kwe, c):
    d = wg_ref.shape[0]

    def we_copy(i):
        return pltpu.make_async_copy(
            we_hbm.at[i // (kwe // 2), pl.ds((i % (kwe // 2)) * f, f), :],
            webuf.at[i % 4], sem_we.at[i % 4])

    def x_copy(j):
        return pltpu.make_async_copy(
            x_hbm.at[pl.ds(j * t, t), :], xbuf.at[j % 4], sem_x.at[j % 4])

    for i in range(4):
        we_copy(i).start()
    for j in range(4):
        x_copy(j).start()

    w4_ref[0:2] = jnp.transpose(wg_ref[...])
    w4_ref[2:4] = jnp.zeros_like(w4_ref[2:4])
    bs0 = jnp.sum(be_ref[0:1, :])
    bs1 = jnp.sum(be_ref[1:2, :])

    for i in range(kwe):
        we_copy(i).wait()
        part = jnp.sum(webuf[i % 4], axis=0, keepdims=True)
        w4_ref[pl.ds(2 + i // (kwe // 2), 1)] += part
        if i + 4 < kwe:
            we_copy(i + 4).start()

    cnt = jnp.float32(0.0)
    for j in range(c):
        x_copy(j).wait()
        lt = jax.lax.dot_general(
            w4_ref[...], xbuf[j % 4],
            dimension_numbers=(((1,), (1,)), ((), ())),
            preferred_element_type=jnp.float32)
        if j + 4 < c:
            x_copy(j + 4).start()
        l0, l1, t0, t1 = lt[0:1], lt[1:2], lt[2:3], lt[3:4]
        e1 = l1 > l0
        m = e1.astype(jnp.float32)
        excl = jnp.dot(m, su_ref[...], preferred_element_type=jnp.float32)
        pos1 = excl + cnt
        slin = (j * t + jax.lax.broadcasted_iota(jnp.int32, (1, t), 1)
                ).astype(jnp.float32)
        pos0 = slin - pos1
        pos = jnp.where(e1, pos1, pos0)
        within = (pos < cap).astype(jnp.float32)
        gate = jax.nn.sigmoid(jnp.abs(l1 - l0))
        tsel = jnp.where(e1, t1 + bs1, t0 + bs0)
        out_ref[0:1, pl.ds(j * t, t)] = gate * within * tsel
        cnt = cnt + jnp.sum(m)


def _lsm_body(v_ref, out_ref):
    v = v_ref[...]
    mx = jnp.max(v, axis=1, keepdims=True)
    lse = jnp.log(jnp.sum(jnp.exp(v - mx), axis=1, keepdims=True)) + mx
    out_ref[...] = v - lse


def kernel(input, wg, We, be):
    B, SEQ, D = input.shape
    E = wg.shape[1]
    S = B * SEQ
    cap = -(-S // E)

    x = input.reshape(S, D)

    F = 512
    KWE = E * (D // F)
    T = 512
    C = S // T

    ii = jax.lax.broadcasted_iota(jnp.int32, (T, T), 0)
    jj = jax.lax.broadcasted_iota(jnp.int32, (T, T), 1)
    su = (ii < jj).astype(jnp.float32)

    val = pl.pallas_call(
        functools.partial(_fused_body, cap=float(cap), t=T, f=F,
                          kwe=KWE, c=C),
        in_specs=[
            pl.BlockSpec(memory_space=pl.ANY),
            pl.BlockSpec(memory_space=pl.ANY),
            pl.BlockSpec((D, E), lambda: (0, 0)),
            pl.BlockSpec((E, D), lambda: (0, 0)),
            pl.BlockSpec((T, T), lambda: (0, 0)),
        ],
        out_specs=pl.BlockSpec((1, S), lambda: (0, 0)),
        out_shape=jax.ShapeDtypeStruct((1, S), jnp.float32),
        scratch_shapes=[
            pltpu.VMEM((4, F, D), jnp.float32),
            pltpu.VMEM((4, T, D), jnp.float32),
            pltpu.VMEM((4, D), jnp.float32),
            pltpu.SemaphoreType.DMA((4,)),
            pltpu.SemaphoreType.DMA((4,)),
        ],
    )(We, x, wg, be, su)

    v = val.reshape(B, SEQ)

    out = pl.pallas_call(
        _lsm_body,
        in_specs=[pl.BlockSpec((B, SEQ), lambda: (0, 0))],
        out_specs=pl.BlockSpec((B, SEQ), lambda: (0, 0)),
        out_shape=jax.ShapeDtypeStruct((B, SEQ), jnp.float32),
    )(v)
    return out
